# Initial kernel scaffold; baseline (speedup 1.0000x reference)
import jax
import jax.numpy as jnp
from jax.experimental import pallas as pl


def kernel(x, day_W, month_W):
    raise NotImplementedError("write your pallas kernel here")


# speedup vs baseline: 3.3856x; 3.3856x over previous
import functools

import jax
import jax.numpy as jnp
from jax import lax
from jax.experimental import pallas as pl
from jax.experimental.pallas import tpu as pltpu
from jax.experimental.pallas import tpu_sc as plsc

N = 16384
D = 128
DAY_ROWS = 32
MONTH_ROWS = 13
MONTH_PAD = 16
TABLE_ROWS = DAY_ROWS * MONTH_PAD

NC = 2
NS = 16
L = 16
NW = NC * NS
ROWS_PER_W = N // NW
CHUNK = 128
NCHUNK = ROWS_PER_W // CHUNK


def _table_body(day_ref, month_ref, out_ref):
    out_ref[...] = day_ref[...][:, None, :] + month_ref[...][None, :, :]


def _build_table(day_W, month_pad):
    out = pl.pallas_call(
        _table_body,
        out_shape=jax.ShapeDtypeStruct((DAY_ROWS, MONTH_PAD, D), jnp.float32),
    )(day_W, month_pad)
    return out.reshape(TABLE_ROWS, D)


_mesh = plsc.VectorSubcoreMesh(
    core_axis_name="c", subcore_axis_name="s", num_cores=NC, num_subcores=NS
)


@functools.partial(
    pl.kernel,
    out_type=jax.ShapeDtypeStruct((N, D), jnp.float32),
    mesh=_mesh,
    scratch_types=[
        pltpu.VMEM((ROWS_PER_W,), jnp.float32),
        pltpu.VMEM((ROWS_PER_W,), jnp.float32),
        pltpu.VMEM((NCHUNK, CHUNK), jnp.int32),
        pltpu.VMEM((NCHUNK, CHUNK, D), jnp.float32),
        pltpu.SemaphoreType.DMA,
        pltpu.SemaphoreType.DMA,
    ],
)
def _sc_lookup(xm_hbm, xd_hbm, table_hbm, out_hbm, xm_v, xd_v, idx_v, rows_v, gsem, wsem):
    wid = lax.axis_index("s") * NC + lax.axis_index("c")
    base = wid * ROWS_PER_W

    pltpu.sync_copy(xm_hbm.at[pl.ds(base, ROWS_PER_W)], xm_v)
    pltpu.sync_copy(xd_hbm.at[pl.ds(base, ROWS_PER_W)], xd_v)

    for i in range(ROWS_PER_W // L):
        m = xm_v[pl.ds(i * L, L)]
        d = xd_v[pl.ds(i * L, L)]
        di = (d * 31.0).astype(jnp.int32)
        mi = (m * 12.0).astype(jnp.int32)
        comb = di * MONTH_PAD + mi
        c, o = divmod(i * L, CHUNK)
        idx_v[c, pl.ds(o, L)] = comb

    gathers = [
        pltpu.async_copy(table_hbm.at[idx_v.at[c]], rows_v.at[c], gsem)
        for c in range(NCHUNK)
    ]
    writes = []
    for c in range(NCHUNK):
        gathers[c].wait()
        writes.append(
            pltpu.async_copy(
                rows_v.at[c], out_hbm.at[pl.ds(base + c * CHUNK, CHUNK)], wsem
            )
        )
    for w in writes:
        w.wait()


def kernel(x, day_W, month_W):
    month_pad = jnp.pad(month_W, ((0, MONTH_PAD - MONTH_ROWS), (0, 0)))
    table = _build_table(day_W, month_pad)
    xm = x[:, 0]
    xd = x[:, 1]
    return _sc_lookup(xm, xd, table)


# speedup vs baseline: 4.3594x; 1.2876x over previous
import functools

import jax
import jax.numpy as jnp
from jax import lax
from jax.experimental import pallas as pl
from jax.experimental.pallas import tpu as pltpu
from jax.experimental.pallas import tpu_sc as plsc

N = 16384
D = 128
DAY_ROWS = 32
MONTH_ROWS = 13
MONTH_PAD = 16
TABLE_ROWS = DAY_ROWS * MONTH_PAD

NC = 2
NS = 16
L = 16
NW = NC * NS
ROWS_PER_W = N // NW
CHUNK = 128
NCHUNK = ROWS_PER_W // CHUNK


def _table_body(day_ref, month_ref, out_ref):
    out_ref[...] = day_ref[...][:, None, :] + month_ref[...][None, :, :]


def _build_table(day_W, month_pad):
    out = pl.pallas_call(
        _table_body,
        out_shape=jax.ShapeDtypeStruct((DAY_ROWS, MONTH_PAD, D), jnp.float32),
    )(day_W, month_pad)
    return out.reshape(TABLE_ROWS, D)


_mesh = plsc.VectorSubcoreMesh(
    core_axis_name="c", subcore_axis_name="s", num_cores=NC, num_subcores=NS
)


@functools.partial(
    pl.kernel,
    out_type=jax.ShapeDtypeStruct((N, D), jnp.float32),
    mesh=_mesh,
    scratch_types=[
        pltpu.VMEM((ROWS_PER_W,), jnp.float32),
        pltpu.VMEM((ROWS_PER_W,), jnp.float32),
        pltpu.VMEM((NCHUNK, CHUNK), jnp.int32),
        pltpu.VMEM((NCHUNK, CHUNK, D), jnp.float32),
        pltpu.VMEM_SHARED((TABLE_ROWS, D), jnp.float32),
        pltpu.SemaphoreType.DMA,
        pltpu.SemaphoreType.DMA,
    ],
)
def _sc_lookup(xm_hbm, xd_hbm, table_hbm, out_hbm, xm_v, xd_v, idx_v, rows_v, table_sh, gsem, wsem):
    sid = lax.axis_index("s")
    wid = sid * NC + lax.axis_index("c")
    base = wid * ROWS_PER_W

    @pl.when(sid == 0)
    def _():
        pltpu.sync_copy(table_hbm, table_sh)

    pltpu.sync_copy(xm_hbm.at[pl.ds(base, ROWS_PER_W)], xm_v)
    pltpu.sync_copy(xd_hbm.at[pl.ds(base, ROWS_PER_W)], xd_v)

    for i in range(ROWS_PER_W // L):
        m = xm_v[pl.ds(i * L, L)]
        d = xd_v[pl.ds(i * L, L)]
        di = (d * 31.0).astype(jnp.int32)
        mi = (m * 12.0).astype(jnp.int32)
        comb = di * MONTH_PAD + mi
        c, o = divmod(i * L, CHUNK)
        idx_v[c, pl.ds(o, L)] = comb

    plsc.subcore_barrier()

    gathers = [
        pltpu.async_copy(table_sh.at[idx_v.at[c]], rows_v.at[c], gsem)
        for c in range(NCHUNK)
    ]
    writes = []
    for c in range(NCHUNK):
        gathers[c].wait()
        writes.append(
            pltpu.async_copy(
                rows_v.at[c], out_hbm.at[pl.ds(base + c * CHUNK, CHUNK)], wsem
            )
        )
    for w in writes:
        w.wait()


def kernel(x, day_W, month_W):
    month_pad = jnp.pad(month_W, ((0, MONTH_PAD - MONTH_ROWS), (0, 0)))
    table = _build_table(day_W, month_pad)
    xm = x[:, 0]
    xd = x[:, 1]
    return _sc_lookup(xm, xd, table)
